# baseline (speedup 1.0000x reference)
import functools

import jax
import jax.numpy as jnp
from jax import lax
from jax.experimental import pallas as pl
from jax.experimental.pallas import tpu as pltpu
from jax.experimental.pallas import tpu_sc as plsc

VOCAB_SIZE = 100000
D_MODEL = 768
CTX_LENGTH = 2048
BATCH = 4

TOTAL = BATCH * CTX_LENGTH
LANES = 16

_info = plsc.get_sparse_core_info()
NC = _info.num_cores
NS = _info.num_subcores
NW = NC * NS
ROWS_PER_W = TOTAL // NW
CHUNK = 64
NCHUNK = ROWS_PER_W // CHUNK


def _emb_kernel(tok_hbm, table_hbm, pos_hbm, out_hbm,
                idx_v, emb_v, pos_v, sem):
    wid = lax.axis_index("s") * NC + lax.axis_index("c")
    base = wid * ROWS_PER_W

    for ch in range(NCHUNK):
        cbase = base + ch * CHUNK
        lbase = lax.rem(cbase, CTX_LENGTH) * D_MODEL
        pltpu.sync_copy(pos_hbm.at[pl.ds(lbase, CHUNK * D_MODEL)], pos_v)
        pltpu.sync_copy(tok_hbm.at[pl.ds(cbase, CHUNK)], idx_v)
        pltpu.async_copy(table_hbm.at[idx_v], emb_v, sem).wait()

        def row_body(r, _):
            for k in range(D_MODEL // LANES):
                off = r * D_MODEL + k * LANES
                pos_v[pl.ds(off, LANES)] = (
                    pos_v[pl.ds(off, LANES)] + emb_v[r, pl.ds(k * LANES, LANES)]
                )
            return 0

        lax.fori_loop(0, CHUNK, row_body, 0)

        pltpu.sync_copy(pos_v, out_hbm.at[pl.ds(cbase * D_MODEL, CHUNK * D_MODEL)])


@functools.partial(jax.jit, static_argnums=())
def _run(tokens_flat, table, pos_flat):
    mesh = plsc.VectorSubcoreMesh(core_axis_name="c", subcore_axis_name="s")
    k = functools.partial(
        pl.kernel,
        mesh=mesh,
        out_type=jax.ShapeDtypeStruct((TOTAL * D_MODEL,), jnp.float32),
        scratch_types=[
            pltpu.VMEM((CHUNK,), jnp.int32),
            pltpu.VMEM((CHUNK, D_MODEL), jnp.float32),
            pltpu.VMEM((CHUNK * D_MODEL,), jnp.float32),
            pltpu.SemaphoreType.DMA,
        ],
    )(_emb_kernel)
    return k(tokens_flat, table, pos_flat)


def kernel(tokens, token_embedding, positional_encoding):
    tokens_flat = tokens.reshape(-1).astype(jnp.int32)
    pos_flat = positional_encoding.reshape(-1)
    out = _run(tokens_flat, token_embedding, pos_flat)
    return out.reshape(BATCH, CTX_LENGTH, D_MODEL)


# speedup vs baseline: 2.5137x; 2.5137x over previous
import functools

import jax
import jax.numpy as jnp
from jax import lax
from jax.experimental import pallas as pl
from jax.experimental.pallas import tpu as pltpu
from jax.experimental.pallas import tpu_sc as plsc

VOCAB_SIZE = 100000
D_MODEL = 768
CTX_LENGTH = 2048
BATCH = 4

TOTAL = BATCH * CTX_LENGTH
LANES = 16

_info = plsc.get_sparse_core_info()
NC = _info.num_cores
NS = _info.num_subcores
NW = NC * NS
LBLK = CTX_LENGTH // NW
CROWS = 32
CH_PER_B = LBLK // CROWS
NCH = BATCH * CH_PER_B
NBUF = 3


def _emb_kernel(tok_hbm, table_hbm, pos_hbm, out_hbm,
                idx_v, pos_v, e0, e1, e2,
                gs0, gs1, gs2, ss0, ss1, ss2, psem):
    wid = lax.axis_index("s") * NC + lax.axis_index("c")
    lbase = wid * LBLK
    embufs = [e0, e1, e2]
    gsems = [gs0, gs1, gs2]
    ssems = [ss0, ss1, ss2]

    pcp = pltpu.async_copy(pos_hbm.at[pl.ds(lbase, LBLK)], pos_v, psem)

    for b in range(BATCH):
        pltpu.sync_copy(tok_hbm.at[pl.ds(b * CTX_LENGTH + lbase, LBLK)],
                        idx_v.at[pl.ds(b * LBLK, LBLK)])

    gathers = {}
    scatters = {}

    def start_gather(c):
        cb = c % NBUF
        gathers[c] = pltpu.async_copy(
            table_hbm.at[idx_v.at[pl.ds(c * CROWS, CROWS)]],
            embufs[cb], gsems[cb])

    start_gather(0)
    start_gather(1)
    pcp.wait()

    for c in range(NCH):
        cb = c % NBUF
        e = embufs[cb]
        gathers[c].wait()

        prow = (c % CH_PER_B) * CROWS

        def row_body(r, _):
            for k in range(D_MODEL // LANES):
                pv = pos_v[prow + r, pl.ds(k * LANES, LANES)]
                plsc.addupdate(e.at[r, pl.ds(k * LANES, LANES)], pv)
            return 0

        lax.fori_loop(0, CROWS, row_body, 0)

        b = c // CH_PER_B
        orow = b * CTX_LENGTH + lbase + prow
        scatters[c] = pltpu.async_copy(
            e, out_hbm.at[pl.ds(orow, CROWS)], ssems[cb])

        if c + 2 < NCH:
            if c - 1 >= 0:
                scatters[c - 1].wait()
            start_gather(c + 2)

    scatters[NCH - 2].wait()
    scatters[NCH - 1].wait()


@jax.jit
def _run(tokens_flat, table, pos_flat):
    mesh = plsc.VectorSubcoreMesh(core_axis_name="c", subcore_axis_name="s")
    k = functools.partial(
        pl.kernel,
        mesh=mesh,
        out_type=jax.ShapeDtypeStruct((TOTAL, D_MODEL), jnp.float32),
        scratch_types=[
            pltpu.VMEM((BATCH * LBLK,), jnp.int32),
            pltpu.VMEM((LBLK, D_MODEL), jnp.float32),
            pltpu.VMEM((CROWS, D_MODEL), jnp.float32),
            pltpu.VMEM((CROWS, D_MODEL), jnp.float32),
            pltpu.VMEM((CROWS, D_MODEL), jnp.float32),
            pltpu.SemaphoreType.DMA,
            pltpu.SemaphoreType.DMA,
            pltpu.SemaphoreType.DMA,
            pltpu.SemaphoreType.DMA,
            pltpu.SemaphoreType.DMA,
            pltpu.SemaphoreType.DMA,
            pltpu.SemaphoreType.DMA,
        ],
    )(_emb_kernel)
    return k(tokens_flat, table, pos_flat)


def kernel(tokens, token_embedding, positional_encoding):
    tokens_flat = tokens.reshape(-1).astype(jnp.int32)
    pos_flat = positional_encoding.reshape(CTX_LENGTH, D_MODEL)
    out = _run(tokens_flat, token_embedding, pos_flat)
    return out.reshape(BATCH, CTX_LENGTH, D_MODEL)


# speedup vs baseline: 2.8580x; 1.1370x over previous
import functools

import jax
import jax.numpy as jnp
from jax import lax
from jax.experimental import pallas as pl
from jax.experimental.pallas import tpu as pltpu
from jax.experimental.pallas import tpu_sc as plsc

VOCAB_SIZE = 100000
D_MODEL = 768
CTX_LENGTH = 2048
BATCH = 4

TOTAL = BATCH * CTX_LENGTH
LANES = 16
KG = D_MODEL // LANES

_info = plsc.get_sparse_core_info()
NC = _info.num_cores
NS = _info.num_subcores
NW = NC * NS
LBLK = CTX_LENGTH // NW
LSUB = 8
CROWS = LSUB * BATCH
NCH = LBLK // LSUB
NBUF = 3


def _emb_kernel(tok_hbm, table_hbm, pos_hbm, out_hbm,
                idx_v, pos_v, e0, e1, e2,
                gs0, gs1, gs2, ss0, ss1, ss2, psem, isem):
    wid = lax.axis_index("s") * NC + lax.axis_index("c")
    lbase = wid * LBLK
    embufs = [e0, e1, e2]
    gsems = [gs0, gs1, gs2]
    ssems = [ss0, ss1, ss2]

    pcp = pltpu.async_copy(pos_hbm.at[pl.ds(lbase, LBLK)], pos_v, psem)

    icps = []
    for j in range(NCH):
        for b in range(BATCH):
            icps.append(pltpu.async_copy(
                tok_hbm.at[pl.ds(b * CTX_LENGTH + lbase + j * LSUB, LSUB)],
                idx_v.at[pl.ds(j * CROWS + b * LSUB, LSUB)], isem))
    for cp in icps:
        cp.wait()

    gathers = {}
    scatters = {}

    def start_gather(c):
        cb = c % NBUF
        gathers[c] = pltpu.async_copy(
            table_hbm.at[idx_v.at[pl.ds(c * CROWS, CROWS)]],
            embufs[cb], gsems[cb])

    start_gather(0)
    start_gather(1)
    pcp.wait()

    for c in range(NCH):
        cb = c % NBUF
        e = embufs[cb]
        gathers[c].wait()

        def row_body(dl, _):
            prow = c * LSUB + dl
            for k in range(KG):
                pv = pos_v[prow, pl.ds(k * LANES, LANES)]
                for b in range(BATCH):
                    plsc.addupdate(
                        e.at[b * LSUB + dl, pl.ds(k * LANES, LANES)], pv)
            return 0

        lax.fori_loop(0, LSUB, row_body, 0)

        scatters[c] = []
        for b in range(BATCH):
            orow = b * CTX_LENGTH + lbase + c * LSUB
            scatters[c].append(pltpu.async_copy(
                e.at[pl.ds(b * LSUB, LSUB)],
                out_hbm.at[pl.ds(orow, LSUB)], ssems[cb]))

        if c + 2 < NCH:
            if c - 1 >= 0:
                for cp in scatters[c - 1]:
                    cp.wait()
            start_gather(c + 2)

    for c in (NCH - 2, NCH - 1):
        for cp in scatters[c]:
            cp.wait()


@jax.jit
def _run(tokens_flat, table, pos2d):
    mesh = plsc.VectorSubcoreMesh(core_axis_name="c", subcore_axis_name="s")
    k = functools.partial(
        pl.kernel,
        mesh=mesh,
        out_type=jax.ShapeDtypeStruct((TOTAL, D_MODEL), jnp.float32),
        scratch_types=[
            pltpu.VMEM((NCH * CROWS,), jnp.int32),
            pltpu.VMEM((LBLK, D_MODEL), jnp.float32),
            pltpu.VMEM((CROWS, D_MODEL), jnp.float32),
            pltpu.VMEM((CROWS, D_MODEL), jnp.float32),
            pltpu.VMEM((CROWS, D_MODEL), jnp.float32),
            pltpu.SemaphoreType.DMA,
            pltpu.SemaphoreType.DMA,
            pltpu.SemaphoreType.DMA,
            pltpu.SemaphoreType.DMA,
            pltpu.SemaphoreType.DMA,
            pltpu.SemaphoreType.DMA,
            pltpu.SemaphoreType.DMA,
            pltpu.SemaphoreType.DMA,
        ],
    )(_emb_kernel)
    return k(tokens_flat, table, pos2d)


def kernel(tokens, token_embedding, positional_encoding):
    tokens_flat = tokens.reshape(-1).astype(jnp.int32)
    pos2d = positional_encoding.reshape(CTX_LENGTH, D_MODEL)
    out = _run(tokens_flat, token_embedding, pos2d)
    return out.reshape(BATCH, CTX_LENGTH, D_MODEL)


# speedup vs baseline: 3.1847x; 1.1143x over previous
import functools

import jax
import jax.numpy as jnp
from jax import lax
from jax.experimental import pallas as pl
from jax.experimental.pallas import tpu as pltpu
from jax.experimental.pallas import tpu_sc as plsc

VOCAB_SIZE = 100000
D_MODEL = 768
CTX_LENGTH = 2048
BATCH = 4

LANES = 16
KG = D_MODEL // LANES

_info = plsc.get_sparse_core_info()
NC = _info.num_cores
NS = _info.num_subcores
NW = NC * NS
LBLK = CTX_LENGTH // NW
LSUB = 8
CROWS = LSUB * BATCH
NCH = LBLK // LSUB
NBUF = 3


def _emb_kernel(tok_hbm, table_hbm, pos_hbm, out_hbm,
                idx_v, pos_v, e0, e1, e2,
                gs0, gs1, gs2, ss0, ss1, ss2, psem, isem):
    wid = lax.axis_index("s") * NC + lax.axis_index("c")
    lbase = wid * LBLK
    embufs = [e0, e1, e2]
    gsems = [gs0, gs1, gs2]
    ssems = [ss0, ss1, ss2]

    pcp = pltpu.async_copy(pos_hbm.at[0, pl.ds(lbase, LBLK)], pos_v, psem)

    icps = []
    for j in range(NCH):
        for b in range(BATCH):
            icps.append(pltpu.async_copy(
                tok_hbm.at[b, pl.ds(lbase + j * LSUB, LSUB)],
                idx_v.at[pl.ds(j * CROWS + b * LSUB, LSUB)], isem))
    for cp in icps:
        cp.wait()

    gathers = {}
    scatters = {}

    def start_gather(c):
        cb = c % NBUF
        gathers[c] = pltpu.async_copy(
            table_hbm.at[idx_v.at[pl.ds(c * CROWS, CROWS)]],
            embufs[cb], gsems[cb])

    start_gather(0)
    start_gather(1)
    pcp.wait()

    for c in range(NCH):
        cb = c % NBUF
        e = embufs[cb]
        gathers[c].wait()

        def col_body(k, _):
            for dl in range(LSUB):
                pv = pos_v[c * LSUB + dl, pl.ds(k * LANES, LANES)]
                for b in range(BATCH):
                    plsc.addupdate(
                        e.at[b * LSUB + dl, pl.ds(k * LANES, LANES)], pv)
            return 0

        lax.fori_loop(0, KG, col_body, 0)

        scatters[c] = []
        for b in range(BATCH):
            scatters[c].append(pltpu.async_copy(
                e.at[pl.ds(b * LSUB, LSUB)],
                out_hbm.at[b, pl.ds(lbase + c * LSUB, LSUB)], ssems[cb]))

        if c + 2 < NCH:
            if c - 1 >= 0:
                for cp in scatters[c - 1]:
                    cp.wait()
            start_gather(c + 2)

    for c in (NCH - 2, NCH - 1):
        for cp in scatters[c]:
            cp.wait()


@jax.jit
def _run(tokens, table, pos):
    mesh = plsc.VectorSubcoreMesh(core_axis_name="c", subcore_axis_name="s")
    k = functools.partial(
        pl.kernel,
        mesh=mesh,
        out_type=jax.ShapeDtypeStruct((BATCH, CTX_LENGTH, D_MODEL), jnp.float32),
        scratch_types=[
            pltpu.VMEM((NCH * CROWS,), jnp.int32),
            pltpu.VMEM((LBLK, D_MODEL), jnp.float32),
            pltpu.VMEM((CROWS, D_MODEL), jnp.float32),
            pltpu.VMEM((CROWS, D_MODEL), jnp.float32),
            pltpu.VMEM((CROWS, D_MODEL), jnp.float32),
            pltpu.SemaphoreType.DMA,
            pltpu.SemaphoreType.DMA,
            pltpu.SemaphoreType.DMA,
            pltpu.SemaphoreType.DMA,
            pltpu.SemaphoreType.DMA,
            pltpu.SemaphoreType.DMA,
            pltpu.SemaphoreType.DMA,
            pltpu.SemaphoreType.DMA,
        ],
    )(_emb_kernel)
    return k(tokens, table, pos)


def kernel(tokens, token_embedding, positional_encoding):
    return _run(tokens.astype(jnp.int32), token_embedding, positional_encoding)


# speedup vs baseline: 3.2340x; 1.0155x over previous
import functools

import jax
import jax.numpy as jnp
from jax import lax
from jax.experimental import pallas as pl
from jax.experimental.pallas import tpu as pltpu
from jax.experimental.pallas import tpu_sc as plsc

VOCAB_SIZE = 100000
D_MODEL = 768
CTX_LENGTH = 2048
BATCH = 4

LANES = 16
KG = D_MODEL // LANES

_info = plsc.get_sparse_core_info()
NC = _info.num_cores
NS = _info.num_subcores
NW = NC * NS
LBLK = CTX_LENGTH // NW
LSUB = 8
CROWS = LSUB * BATCH
NCH = LBLK // LSUB
NBUF = 3


def _emb_kernel(tok_hbm, table_hbm, pos_hbm, out_hbm,
                idx_v, pos_v, e0, e1, e2,
                gs0, gs1, gs2, ss0, ss1, ss2, psem, isem):
    wid = lax.axis_index("s") * NC + lax.axis_index("c")
    lbase = wid * LBLK
    embufs = [e0, e1, e2]
    gsems = [gs0, gs1, gs2]
    ssems = [ss0, ss1, ss2]

    pcp = pltpu.async_copy(pos_hbm.at[0, pl.ds(lbase, LBLK)], pos_v, psem)

    idx_cps = {}

    def load_idx(j):
        idx_cps[j] = [pltpu.async_copy(
            tok_hbm.at[b, pl.ds(lbase + j * LSUB, LSUB)],
            idx_v.at[pl.ds(j * CROWS + b * LSUB, LSUB)], isem)
            for b in range(BATCH)]

    gathers = {}
    scatters = {}

    def start_gather(c):
        cb = c % NBUF
        for cp in idx_cps[c]:
            cp.wait()
        gathers[c] = pltpu.async_copy(
            table_hbm.at[idx_v.at[pl.ds(c * CROWS, CROWS)]],
            embufs[cb], gsems[cb])

    load_idx(0)
    load_idx(1)
    start_gather(0)
    start_gather(1)
    for j in range(2, NCH):
        load_idx(j)
    pcp.wait()

    for c in range(NCH):
        cb = c % NBUF
        e = embufs[cb]
        gathers[c].wait()

        @plsc.parallel_loop(0, KG)
        def col_body(k):
            for dl in range(LSUB):
                pv = pos_v[c * LSUB + dl, pl.ds(k * LANES, LANES)]
                for b in range(BATCH):
                    plsc.addupdate(
                        e.at[b * LSUB + dl, pl.ds(k * LANES, LANES)], pv)

        scatters[c] = []
        for b in range(BATCH):
            scatters[c].append(pltpu.async_copy(
                e.at[pl.ds(b * LSUB, LSUB)],
                out_hbm.at[b, pl.ds(lbase + c * LSUB, LSUB)], ssems[cb]))

        if c + 2 < NCH:
            if c - 1 >= 0:
                for cp in scatters[c - 1]:
                    cp.wait()
            start_gather(c + 2)

    for c in (NCH - 2, NCH - 1):
        for cp in scatters[c]:
            cp.wait()


@jax.jit
def _run(tokens, table, pos):
    mesh = plsc.VectorSubcoreMesh(core_axis_name="c", subcore_axis_name="s")
    k = functools.partial(
        pl.kernel,
        mesh=mesh,
        out_type=jax.ShapeDtypeStruct((BATCH, CTX_LENGTH, D_MODEL), jnp.float32),
        scratch_types=[
            pltpu.VMEM((NCH * CROWS,), jnp.int32),
            pltpu.VMEM((LBLK, D_MODEL), jnp.float32),
            pltpu.VMEM((CROWS, D_MODEL), jnp.float32),
            pltpu.VMEM((CROWS, D_MODEL), jnp.float32),
            pltpu.VMEM((CROWS, D_MODEL), jnp.float32),
            pltpu.SemaphoreType.DMA,
            pltpu.SemaphoreType.DMA,
            pltpu.SemaphoreType.DMA,
            pltpu.SemaphoreType.DMA,
            pltpu.SemaphoreType.DMA,
            pltpu.SemaphoreType.DMA,
            pltpu.SemaphoreType.DMA,
            pltpu.SemaphoreType.DMA,
        ],
    )(_emb_kernel)
    return k(tokens, table, pos)


def kernel(tokens, token_embedding, positional_encoding):
    return _run(tokens.astype(jnp.int32), token_embedding, positional_encoding)


# speedup vs baseline: 3.2430x; 1.0028x over previous
import functools

import jax
import jax.numpy as jnp
from jax import lax
from jax.experimental import pallas as pl
from jax.experimental.pallas import tpu as pltpu
from jax.experimental.pallas import tpu_sc as plsc

VOCAB_SIZE = 100000
D_MODEL = 768
CTX_LENGTH = 2048
BATCH = 4

LANES = 16
KG = D_MODEL // LANES

_info = plsc.get_sparse_core_info()
NC = _info.num_cores
NS = _info.num_subcores
NW = NC * NS
LBLK = CTX_LENGTH // NW
LSUB = 8
CROWS = LSUB * BATCH
NCH = LBLK // LSUB
NBUF = 3


def _emb_kernel(tok_hbm, table_hbm, pos_hbm, out_hbm,
                idx_v, pos_v, e0, e1, e2,
                gs0, gs1, gs2, ss0, ss1, ss2, psem, isem):
    wid = lax.axis_index("s") * NC + lax.axis_index("c")
    lbase = wid * LBLK
    embufs = [e0, e1, e2]
    gsems = [gs0, gs1, gs2]
    ssems = [ss0, ss1, ss2]

    pcp = pltpu.async_copy(pos_hbm.at[0, pl.ds(lbase, LBLK)], pos_v, psem)

    idx_cps = {}

    def load_idx(j):
        idx_cps[j] = [pltpu.async_copy(
            tok_hbm.at[b, pl.ds(lbase + j * LSUB, LSUB)],
            idx_v.at[j, 0, pl.ds(b * LSUB, LSUB)], isem)
            for b in range(BATCH)]

    gathers = {}
    scatters = {}

    def start_gather(c):
        cb = c % NBUF
        for cp in idx_cps[c]:
            cp.wait()
        gathers[c] = pltpu.async_copy(
            table_hbm.at[idx_v.at[c, 0]], embufs[cb], gsems[cb])

    load_idx(0)
    load_idx(1)
    start_gather(0)
    start_gather(1)
    for j in range(2, NCH):
        load_idx(j)
    pcp.wait()

    for c in range(NCH):
        cb = c % NBUF
        e = embufs[cb]
        gathers[c].wait()

        @plsc.parallel_loop(0, KG)
        def col_body(k):
            for dl in range(LSUB):
                pv = pos_v[c * LSUB + dl, pl.ds(k * LANES, LANES)]
                for b in range(BATCH):
                    plsc.addupdate(
                        e.at[b * LSUB + dl, pl.ds(k * LANES, LANES)], pv)

        scatters[c] = []
        for b in range(BATCH):
            scatters[c].append(pltpu.async_copy(
                e.at[pl.ds(b * LSUB, LSUB)],
                out_hbm.at[b, pl.ds(lbase + c * LSUB, LSUB)], ssems[cb]))

        if c + 2 < NCH:
            if c - 1 >= 0:
                for cp in scatters[c - 1]:
                    cp.wait()
            start_gather(c + 2)

    for c in (NCH - 2, NCH - 1):
        for cp in scatters[c]:
            cp.wait()


@jax.jit
def _run(tokens, table, pos):
    mesh = plsc.VectorSubcoreMesh(core_axis_name="c", subcore_axis_name="s")
    k = functools.partial(
        pl.kernel,
        mesh=mesh,
        out_type=jax.ShapeDtypeStruct((BATCH, CTX_LENGTH, D_MODEL), jnp.float32),
        scratch_types=[
            pltpu.VMEM((NCH, 1, CROWS), jnp.int32),
            pltpu.VMEM((LBLK, D_MODEL), jnp.float32),
            pltpu.VMEM((CROWS, D_MODEL), jnp.float32),
            pltpu.VMEM((CROWS, D_MODEL), jnp.float32),
            pltpu.VMEM((CROWS, D_MODEL), jnp.float32),
            pltpu.SemaphoreType.DMA,
            pltpu.SemaphoreType.DMA,
            pltpu.SemaphoreType.DMA,
            pltpu.SemaphoreType.DMA,
            pltpu.SemaphoreType.DMA,
            pltpu.SemaphoreType.DMA,
            pltpu.SemaphoreType.DMA,
            pltpu.SemaphoreType.DMA,
        ],
    )(_emb_kernel)
    return k(tokens, table, pos)


def kernel(tokens, token_embedding, positional_encoding):
    return _run(tokens.astype(jnp.int32), token_embedding, positional_encoding)


# speedup vs baseline: 3.2500x; 1.0022x over previous
import functools

import jax
import jax.numpy as jnp
from jax import lax
from jax.experimental import pallas as pl
from jax.experimental.pallas import tpu as pltpu
from jax.experimental.pallas import tpu_sc as plsc

VOCAB_SIZE = 100000
D_MODEL = 768
CTX_LENGTH = 2048
BATCH = 4

LANES = 16
KG = D_MODEL // LANES

_info = plsc.get_sparse_core_info()
NC = _info.num_cores
NS = _info.num_subcores
NW = NC * NS
LBLK = CTX_LENGTH // NW
LSUB = 8
CROWS = LSUB * BATCH
NCH = LBLK // LSUB
NBUF = 3


def _emb_kernel(tok_hbm, table_hbm, pos_hbm, out_hbm,
                idx_v, pos_v, e0, e1, e2,
                gs0, gs1, gs2, ss0, ss1, ss2, psem, isem):
    wid = lax.axis_index("s") * NC + lax.axis_index("c")
    lbase = wid * LBLK
    embufs = [e0, e1, e2]
    gsems = [gs0, gs1, gs2]
    ssems = [ss0, ss1, ss2]

    pcp = pltpu.async_copy(pos_hbm.at[0, pl.ds(lbase, LBLK)], pos_v, psem)

    idx_cps = {}

    def load_idx(j):
        idx_cps[j] = [pltpu.async_copy(
            tok_hbm.at[b, pl.ds(lbase + j * LSUB, LSUB)],
            idx_v.at[j, 0, pl.ds(b * LSUB, LSUB)], isem)
            for b in range(BATCH)]

    gathers = {}
    scatters = {}

    def start_gather(c):
        cb = c % NBUF
        for cp in idx_cps[c]:
            cp.wait()
        gathers[c] = pltpu.async_copy(
            table_hbm.at[idx_v.at[c, 0]], embufs[cb], gsems[cb])

    load_idx(0)
    load_idx(1)
    start_gather(0)
    start_gather(1)
    for j in range(2, NCH):
        load_idx(j)
    pcp.wait()

    for c in range(NCH):
        cb = c % NBUF
        e = embufs[cb]
        gathers[c].wait()

        @plsc.parallel_loop(0, KG)
        def col_body(k):
            for dl in range(LSUB):
                pv = pos_v[c * LSUB + dl, pl.ds(k * LANES, LANES)]
                for b in range(BATCH):
                    plsc.addupdate(
                        e.at[b * LSUB + dl, pl.ds(k * LANES, LANES)], pv)

        if c + 2 < NCH:
            if c - 1 >= 0:
                for cp in scatters[c - 1]:
                    cp.wait()
            start_gather(c + 2)

        scatters[c] = []
        for b in range(BATCH):
            scatters[c].append(pltpu.async_copy(
                e.at[pl.ds(b * LSUB, LSUB)],
                out_hbm.at[b, pl.ds(lbase + c * LSUB, LSUB)], ssems[cb]))

    for c in (NCH - 2, NCH - 1):
        for cp in scatters[c]:
            cp.wait()


@jax.jit
def _run(tokens, table, pos):
    mesh = plsc.VectorSubcoreMesh(core_axis_name="c", subcore_axis_name="s")
    k = functools.partial(
        pl.kernel,
        mesh=mesh,
        out_type=jax.ShapeDtypeStruct((BATCH, CTX_LENGTH, D_MODEL), jnp.float32),
        scratch_types=[
            pltpu.VMEM((NCH, 1, CROWS), jnp.int32),
            pltpu.VMEM((LBLK, D_MODEL), jnp.float32),
            pltpu.VMEM((CROWS, D_MODEL), jnp.float32),
            pltpu.VMEM((CROWS, D_MODEL), jnp.float32),
            pltpu.VMEM((CROWS, D_MODEL), jnp.float32),
            pltpu.SemaphoreType.DMA,
            pltpu.SemaphoreType.DMA,
            pltpu.SemaphoreType.DMA,
            pltpu.SemaphoreType.DMA,
            pltpu.SemaphoreType.DMA,
            pltpu.SemaphoreType.DMA,
            pltpu.SemaphoreType.DMA,
            pltpu.SemaphoreType.DMA,
        ],
    )(_emb_kernel)
    return k(tokens, table, pos)


def kernel(tokens, token_embedding, positional_encoding):
    return _run(tokens.astype(jnp.int32), token_embedding, positional_encoding)
